# baseline (speedup 1.0000x reference)
import functools

import jax
import jax.numpy as jnp
from jax import lax
from jax.experimental import pallas as pl
from jax.experimental.pallas import tpu as pltpu
from jax.experimental.pallas import tpu_sc as plsc

NC = 2
NS = 16
LANES = 16
CHUNK = 128
PHASES = 4
GRAN = 8 * PHASES
SC0_FRAC = 0.8


def _pad_up(v, m):
    return -(-v // m) * m


def _split_rows(e):
    total_pw = _pad_up(_pad_up(e, NS * CHUNK) // (NS * CHUNK), GRAN)
    pw0 = int(round(total_pw * SC0_FRAC / GRAN)) * GRAN
    pw0 = min(max(pw0, GRAN), total_pw - GRAN)
    pw1 = total_pw - pw0
    return pw0, pw1


def _make_deg_kernel(n_pad, pw0, pw1):
    mesh = plsc.VectorSubcoreMesh(core_axis_name="c", subcore_axis_name="s")
    nw = NC * NS

    @functools.partial(
        pl.kernel,
        out_type=jax.ShapeDtypeStruct((nw, n_pad), jnp.float32),
        mesh=mesh,
        scratch_types=[
            pltpu.VMEM((n_pad,), jnp.float32),
            pltpu.VMEM((pw0, CHUNK), jnp.int32),
        ],
        compiler_params=pltpu.CompilerParams(needs_layout_passes=False),
    )
    def deg_kernel(dst_hbm, out_hbm, hist, dstv):
        cid = lax.axis_index("c")
        sid = lax.axis_index("s")
        wid = sid * NC + cid
        my_pw = jnp.where(cid == 0, pw0, pw1)
        row_base = pl.multiple_of(
            jnp.where(cid == 0, sid * pw0, NS * pw0 + sid * pw1), 8
        )
        z16 = jnp.zeros((LANES,), jnp.float32)

        def zb(i, c):
            hist[pl.ds(i * LANES, LANES)] = z16
            return c

        lax.fori_loop(0, n_pad // LANES, zb, 0)
        pltpu.sync_copy(dst_hbm.at[pl.ds(row_base, pw0)], dstv)
        ones16 = jnp.ones((LANES,), jnp.float32)
        groups = CHUNK // LANES

        def eb(g, c):
            j = g // groups
            l = g - j * groups
            idx = dstv[j, pl.ds(l * LANES, LANES)]
            plsc.addupdate_scatter(hist, [idx], ones16)
            return c

        lax.fori_loop(0, my_pw * groups, eb, 0)
        pltpu.sync_copy(hist, out_hbm.at[wid])

    return deg_kernel


def _make_scatter_kernel(n_nodes, n_pad, pw, h):
    mesh = plsc.VectorSubcoreMesh(
        core_axis_name="c", subcore_axis_name="s", num_cores=1
    )
    rows_pt = n_pad // NS
    depth = 2
    ppw = pw // PHASES
    assert pw % PHASES == 0 and ppw % depth == 0 and ppw // depth >= 2

    @functools.partial(
        pl.kernel,
        out_type=jax.ShapeDtypeStruct((n_pad, h), jnp.float32),
        mesh=mesh,
        scratch_types=[
            pltpu.VMEM((ppw, CHUNK), jnp.int32),
            pltpu.VMEM((ppw, CHUNK), jnp.int32),
            [pltpu.VMEM((CHUNK, h), jnp.float32)] * depth,
            pltpu.VMEM_SHARED((n_pad, h), jnp.float32),
            [pltpu.SemaphoreType.DMA] * depth,
            [pltpu.SemaphoreType.DMA] * depth,
        ],
        compiler_params=pltpu.CompilerParams(needs_layout_passes=False),
    )
    def scat_kernel(m_hbm, src_hbm, dst_hbm, out_hbm, srcv, dstv, bufs, acc,
                    gsem, ssem):
        sid = lax.axis_index("s")
        my_ppw = ppw
        row_base = sid * pw
        z16 = jnp.zeros((LANES,), jnp.float32)
        per_row = h // LANES

        def zb(i, c):
            r = i // per_row
            q = i - r * per_row
            bufs[0][r, pl.ds(q * LANES, LANES)] = z16
            return c

        lax.fori_loop(0, CHUNK * per_row, zb, 0)
        for k in range(rows_pt // CHUNK):
            pltpu.sync_copy(
                bufs[0], acc.at[pl.ds(sid * rows_pt + k * CHUNK, CHUNK)]
            )
        plsc.subcore_barrier()

        def fire_gather(j, b):
            pltpu.async_copy(m_hbm.at[srcv.at[j]], bufs[b], gsem[b])

        def drain_gather(b):
            pltpu.make_async_copy(m_hbm.at[srcv.at[0]], bufs[b], gsem[b]).wait()

        def fire_scatter(j, b):
            pltpu.async_copy(bufs[b], acc.at[dstv.at[j]], ssem[b], add=True)

        def drain_scatter(b):
            pltpu.make_async_copy(bufs[b], acc.at[dstv.at[0]], ssem[b]).wait()

        for p in range(PHASES):
            base = pl.multiple_of(row_base + p * my_ppw, 8)
            pltpu.sync_copy(src_hbm.at[pl.ds(base, ppw)], srcv)
            pltpu.sync_copy(dst_hbm.at[pl.ds(base, ppw)], dstv)

            for b in range(depth):
                fire_gather(b, b)

            def body(i, c):
                j0 = i * depth
                for b in range(depth):
                    drain_gather(b)
                    fire_scatter(j0 + b, b)
                for b in range(depth):
                    drain_scatter(b)
                    fire_gather(j0 + depth + b, b)
                return c

            lax.fori_loop(0, my_ppw // depth - 1, body, 0)
            j0 = my_ppw - depth
            for b in range(depth):
                drain_gather(b)
                fire_scatter(j0 + b, b)
            for b in range(depth):
                drain_scatter(b)

        plsc.subcore_barrier()
        pltpu.sync_copy(
            acc.at[pl.ds(sid * rows_pt, rows_pt)],
            out_hbm.at[pl.ds(sid * rows_pt, rows_pt)],
        )

    return scat_kernel


def _tc_prep(dp, x, w1):
    n, _ = x.shape
    hh = w1.shape[0]
    nw = dp.shape[0]

    def body(dp_ref, x_ref, w_ref, m_ref, dinv_ref):
        ones = jnp.ones((nw, 1), jnp.float32)
        deg = lax.dot_general(
            dp_ref[:, :n], ones, (((0,), (0,)), ((), ())),
            preferred_element_type=jnp.float32,
        )
        dinv = lax.rsqrt(deg + 1.0)
        m = lax.dot_general(
            x_ref[...], w_ref[...], (((1,), (1,)), ((), ())),
            preferred_element_type=jnp.float32,
        )
        m_ref[...] = m * dinv
        dinv_ref[...] = dinv

    return pl.pallas_call(
        body,
        out_shape=(
            jax.ShapeDtypeStruct((n, hh), jnp.float32),
            jax.ShapeDtypeStruct((n, 1), jnp.float32),
        ),
    )(dp, x, w1)


def _norm_act(acc_ref, m_ref, dinv_ref, b_ref, g_ref, beta_ref, n):
    agg = acc_ref[:n, :] + m_ref[...]
    pre = agg * dinv_ref[...] + b_ref[...]
    mu = jnp.mean(pre, axis=1, keepdims=True)
    var = jnp.mean((pre - mu) ** 2, axis=1, keepdims=True)
    y = (pre - mu) * lax.rsqrt(var + 1e-5) * g_ref[...] + beta_ref[...]
    return jnp.where(y > 0, y, jnp.exp(jnp.minimum(y, 0.0)) - 1.0)


def _tc_mid(acc, m, dinv, b, g, beta, w2):
    n, hh = m.shape

    def body(acc_ref, m_ref, dinv_ref, b_ref, g_ref, beta_ref, w_ref, out_ref):
        hcur = _norm_act(acc_ref, m_ref, dinv_ref, b_ref, g_ref, beta_ref, n)
        m2 = lax.dot_general(
            hcur, w_ref[...], (((1,), (1,)), ((), ())),
            preferred_element_type=jnp.float32,
        )
        out_ref[...] = m2 * dinv_ref[...]

    return pl.pallas_call(
        body,
        out_shape=jax.ShapeDtypeStruct((n, hh), jnp.float32),
    )(acc, m, dinv, b, g, beta, w2)


def _tc_final(acc, m, dinv, b, g, beta, batch2d, wc, bc, n_seg):
    n, hh = m.shape
    c = wc.shape[0]

    def body(acc_ref, m_ref, dinv_ref, b_ref, g_ref, beta_ref, batch_ref,
             wc_ref, bc_ref, out_ref):
        hcur = _norm_act(acc_ref, m_ref, dinv_ref, b_ref, g_ref, beta_ref, n)
        seg_iota = lax.broadcasted_iota(jnp.int32, (n, n_seg), 1)
        onehot = (batch_ref[...] == seg_iota).astype(jnp.float32)
        sums = lax.dot_general(
            onehot, hcur, (((0,), (0,)), ((), ())),
            preferred_element_type=jnp.float32,
        )
        cnt = lax.dot_general(
            onehot, jnp.ones((n, 1), jnp.float32), (((0,), (0,)), ((), ())),
            preferred_element_type=jnp.float32,
        )
        gpool = sums / jnp.maximum(cnt, 1.0)
        out_ref[...] = lax.dot_general(
            gpool, wc_ref[...], (((1,), (1,)), ((), ())),
            preferred_element_type=jnp.float32,
        ) + bc_ref[...]

    return pl.pallas_call(
        body,
        out_shape=jax.ShapeDtypeStruct((n_seg, c), jnp.float32),
    )(acc, m, dinv, b, g, beta, batch2d, wc, bc)


def kernel(x, edge_index, batch, W1, b1, W2, b2, ln1_g, ln1_b, ln2_g, ln2_b,
           Wc, bc):
    n, d = x.shape
    hh = W1.shape[0]
    e = edge_index.shape[1]
    n_seg = 64

    pw0, pw1 = _split_rows(e)
    tot_rows = NS * (pw0 + pw1)
    pad_rows = tot_rows + pw0
    ep = pad_rows * CHUNK
    n_pad = _pad_up(n + 1, NS * CHUNK)

    src = edge_index[0]
    dst = edge_index[1]
    src_p = jnp.concatenate([src, jnp.zeros((ep - e,), jnp.int32)])
    dst_p = jnp.concatenate([dst, jnp.full((ep - e,), n, jnp.int32)])
    src2d = src_p.reshape(pad_rows, CHUNK)
    dst2d = dst_p.reshape(pad_rows, CHUNK)

    deg_kernel = _make_deg_kernel(n_pad, pw0, pw1)
    scat_kernel = _make_scatter_kernel(n, n_pad, pw0 + pw1, hh)

    dp = deg_kernel(dst2d)
    m1s, dinv = _tc_prep(dp, x, W1)
    acc1 = scat_kernel(m1s, src2d, dst2d)
    m2s = _tc_mid(acc1, m1s, dinv, b1.reshape(1, hh), ln1_g.reshape(1, hh),
                  ln1_b.reshape(1, hh), W2)
    acc2 = scat_kernel(m2s, src2d, dst2d)
    logits = _tc_final(acc2, m2s, dinv, b2.reshape(1, hh),
                       ln2_g.reshape(1, hh), ln2_b.reshape(1, hh),
                       batch.reshape(n, 1), Wc, bc.reshape(1, -1), n_seg)
    return logits


# speedup vs baseline: 1.4749x; 1.4749x over previous
import functools

import jax
import jax.numpy as jnp
from jax import lax
from jax.experimental import pallas as pl
from jax.experimental.pallas import tpu as pltpu
from jax.experimental.pallas import tpu_sc as plsc

NC = 2
NS = 16
LANES = 16
CHUNK = 128
PHASES = 4
GRAN = 8 * PHASES
SC0_FRAC = 0.8


def _pad_up(v, m):
    return -(-v // m) * m


def _split_rows(e):
    total_pw = _pad_up(_pad_up(e, NS * CHUNK) // (NS * CHUNK), GRAN)
    pw0 = int(round(total_pw * SC0_FRAC / GRAN)) * GRAN
    pw0 = min(max(pw0, GRAN), total_pw - GRAN)
    pw1 = total_pw - pw0
    return pw0, pw1


def _make_deg_kernel(n_pad, pw0, pw1):
    mesh = plsc.VectorSubcoreMesh(core_axis_name="c", subcore_axis_name="s")
    nw = NC * NS

    @functools.partial(
        pl.kernel,
        out_type=jax.ShapeDtypeStruct((nw, n_pad), jnp.float32),
        mesh=mesh,
        scratch_types=[
            pltpu.VMEM((n_pad,), jnp.float32),
            pltpu.VMEM((pw0, CHUNK), jnp.int32),
        ],
        compiler_params=pltpu.CompilerParams(needs_layout_passes=False),
    )
    def deg_kernel(dst_hbm, out_hbm, hist, dstv):
        cid = lax.axis_index("c")
        sid = lax.axis_index("s")
        wid = sid * NC + cid
        my_pw = jnp.where(cid == 0, pw0, pw1)
        row_base = pl.multiple_of(
            jnp.where(cid == 0, sid * pw0, NS * pw0 + sid * pw1), 8
        )
        z16 = jnp.zeros((LANES,), jnp.float32)

        def zb(i, c):
            hist[pl.ds(i * LANES, LANES)] = z16
            return c

        lax.fori_loop(0, n_pad // LANES, zb, 0)
        pltpu.sync_copy(dst_hbm.at[pl.ds(row_base, pw0)], dstv)
        ones16 = jnp.ones((LANES,), jnp.float32)
        groups = CHUNK // LANES

        def eb(g, c):
            j = g // groups
            l = g - j * groups
            idx = dstv[j, pl.ds(l * LANES, LANES)]
            plsc.addupdate_scatter(hist, [idx], ones16)
            return c

        lax.fori_loop(0, my_pw * groups, eb, 0)
        pltpu.sync_copy(hist, out_hbm.at[wid])

    return deg_kernel


def _make_scatter_kernel(n_nodes, n_pad, pw0, pw1, h):
    mesh = plsc.VectorSubcoreMesh(core_axis_name="c", subcore_axis_name="s")
    rows_pt = n_pad // NS
    depth = 2
    ppw = pw0 // PHASES
    ppw1 = pw1 // PHASES
    assert ppw1 >= depth and ppw % depth == 0 and ppw1 % depth == 0

    @functools.partial(
        pl.kernel,
        out_type=jax.ShapeDtypeStruct((NC, n_pad, h), jnp.float32),
        mesh=mesh,
        scratch_types=[
            pltpu.VMEM((ppw, CHUNK), jnp.int32),
            pltpu.VMEM((ppw, CHUNK), jnp.int32),
            [pltpu.VMEM((CHUNK, h), jnp.float32)] * depth,
            pltpu.VMEM_SHARED((n_pad, h), jnp.float32),
            [pltpu.SemaphoreType.DMA] * depth,
            [pltpu.SemaphoreType.DMA] * depth,
        ],
        compiler_params=pltpu.CompilerParams(needs_layout_passes=False),
    )
    def scat_kernel(m_hbm, src_hbm, dst_hbm, out_hbm, srcv, dstv, bufs, acc,
                    gsem, ssem):
        cid = lax.axis_index("c")
        sid = lax.axis_index("s")
        my_ppw = jnp.where(cid == 0, ppw, ppw1)
        row_base = jnp.where(cid == 0, sid * pw0, NS * pw0 + sid * pw1)
        z16 = jnp.zeros((LANES,), jnp.float32)
        per_row = h // LANES

        def zb(i, c):
            r = i // per_row
            q = i - r * per_row
            bufs[0][r, pl.ds(q * LANES, LANES)] = z16
            return c

        lax.fori_loop(0, CHUNK * per_row, zb, 0)
        for k in range(rows_pt // CHUNK):
            pltpu.sync_copy(
                bufs[0], acc.at[pl.ds(sid * rows_pt + k * CHUNK, CHUNK)]
            )
        plsc.subcore_barrier()

        def fire_gather(j, b):
            pltpu.async_copy(m_hbm.at[srcv.at[j]], bufs[b], gsem[b])

        def drain_gather(b):
            pltpu.make_async_copy(m_hbm.at[srcv.at[0]], bufs[b], gsem[b]).wait()

        def fire_scatter(j, b):
            pltpu.async_copy(bufs[b], acc.at[dstv.at[j]], ssem[b], add=True)

        def drain_scatter(b):
            pltpu.make_async_copy(bufs[b], acc.at[dstv.at[0]], ssem[b]).wait()

        for p in range(PHASES):
            base = pl.multiple_of(row_base + p * my_ppw, 8)
            pltpu.sync_copy(src_hbm.at[pl.ds(base, ppw)], srcv)
            pltpu.sync_copy(dst_hbm.at[pl.ds(base, ppw)], dstv)

            for b in range(depth):
                fire_gather(b, b)

            def body(i, c):
                j0 = i * depth
                for b in range(depth):
                    drain_gather(b)
                    fire_scatter(j0 + b, b)
                for b in range(depth):
                    drain_scatter(b)
                    fire_gather(j0 + depth + b, b)
                return c

            lax.fori_loop(0, my_ppw // depth - 1, body, 0)
            j0 = my_ppw - depth
            for b in range(depth):
                drain_gather(b)
                fire_scatter(j0 + b, b)
            for b in range(depth):
                drain_scatter(b)

        plsc.subcore_barrier()
        pltpu.sync_copy(
            acc.at[pl.ds(sid * rows_pt, rows_pt)],
            out_hbm.at[cid, pl.ds(sid * rows_pt, rows_pt)],
        )

    return scat_kernel


def _tc_prep(dp, x, w1):
    n, _ = x.shape
    hh = w1.shape[0]
    nw = dp.shape[0]

    def body(dp_ref, x_ref, w_ref, m_ref, dinv_ref):
        ones = jnp.ones((nw, 1), jnp.float32)
        deg = lax.dot_general(
            dp_ref[:, :n], ones, (((0,), (0,)), ((), ())),
            preferred_element_type=jnp.float32,
        )
        dinv = lax.rsqrt(deg + 1.0)
        m = lax.dot_general(
            x_ref[...], w_ref[...], (((1,), (1,)), ((), ())),
            preferred_element_type=jnp.float32,
        )
        m_ref[...] = m * dinv
        dinv_ref[...] = dinv

    return pl.pallas_call(
        body,
        out_shape=(
            jax.ShapeDtypeStruct((n, hh), jnp.float32),
            jax.ShapeDtypeStruct((n, 1), jnp.float32),
        ),
    )(dp, x, w1)


def _norm_act(acc_ref, m_ref, dinv_ref, b_ref, g_ref, beta_ref, n):
    agg = acc_ref[0, :n, :] + acc_ref[1, :n, :] + m_ref[...]
    pre = agg * dinv_ref[...] + b_ref[...]
    mu = jnp.mean(pre, axis=1, keepdims=True)
    var = jnp.mean((pre - mu) ** 2, axis=1, keepdims=True)
    y = (pre - mu) * lax.rsqrt(var + 1e-5) * g_ref[...] + beta_ref[...]
    return jnp.where(y > 0, y, jnp.exp(jnp.minimum(y, 0.0)) - 1.0)


def _tc_mid(acc, m, dinv, b, g, beta, w2):
    n, hh = m.shape

    def body(acc_ref, m_ref, dinv_ref, b_ref, g_ref, beta_ref, w_ref, out_ref):
        hcur = _norm_act(acc_ref, m_ref, dinv_ref, b_ref, g_ref, beta_ref, n)
        m2 = lax.dot_general(
            hcur, w_ref[...], (((1,), (1,)), ((), ())),
            preferred_element_type=jnp.float32,
        )
        out_ref[...] = m2 * dinv_ref[...]

    return pl.pallas_call(
        body,
        out_shape=jax.ShapeDtypeStruct((n, hh), jnp.float32),
    )(acc, m, dinv, b, g, beta, w2)


def _tc_final(acc, m, dinv, b, g, beta, batch2d, wc, bc, n_seg):
    n, hh = m.shape
    c = wc.shape[0]

    def body(acc_ref, m_ref, dinv_ref, b_ref, g_ref, beta_ref, batch_ref,
             wc_ref, bc_ref, out_ref):
        hcur = _norm_act(acc_ref, m_ref, dinv_ref, b_ref, g_ref, beta_ref, n)
        seg_iota = lax.broadcasted_iota(jnp.int32, (n, n_seg), 1)
        onehot = (batch_ref[...] == seg_iota).astype(jnp.float32)
        sums = lax.dot_general(
            onehot, hcur, (((0,), (0,)), ((), ())),
            preferred_element_type=jnp.float32,
        )
        cnt = lax.dot_general(
            onehot, jnp.ones((n, 1), jnp.float32), (((0,), (0,)), ((), ())),
            preferred_element_type=jnp.float32,
        )
        gpool = sums / jnp.maximum(cnt, 1.0)
        out_ref[...] = lax.dot_general(
            gpool, wc_ref[...], (((1,), (1,)), ((), ())),
            preferred_element_type=jnp.float32,
        ) + bc_ref[...]

    return pl.pallas_call(
        body,
        out_shape=jax.ShapeDtypeStruct((n_seg, c), jnp.float32),
    )(acc, m, dinv, b, g, beta, batch2d, wc, bc)


def kernel(x, edge_index, batch, W1, b1, W2, b2, ln1_g, ln1_b, ln2_g, ln2_b,
           Wc, bc):
    n, d = x.shape
    hh = W1.shape[0]
    e = edge_index.shape[1]
    n_seg = 64

    pw0, pw1 = _split_rows(e)
    tot_rows = NS * (pw0 + pw1)
    pad_rows = tot_rows + pw0
    ep = pad_rows * CHUNK
    n_pad = _pad_up(n + 1, NS * CHUNK)

    src = edge_index[0]
    dst = edge_index[1]
    src_p = jnp.concatenate([src, jnp.zeros((ep - e,), jnp.int32)])
    dst_p = jnp.concatenate([dst, jnp.full((ep - e,), n, jnp.int32)])
    src2d = src_p.reshape(pad_rows, CHUNK)
    dst2d = dst_p.reshape(pad_rows, CHUNK)

    deg_kernel = _make_deg_kernel(n_pad, pw0, pw1)
    scat_kernel = _make_scatter_kernel(n, n_pad, pw0, pw1, hh)

    dp = deg_kernel(dst2d)
    m1s, dinv = _tc_prep(dp, x, W1)
    acc1 = scat_kernel(m1s, src2d, dst2d)
    m2s = _tc_mid(acc1, m1s, dinv, b1.reshape(1, hh), ln1_g.reshape(1, hh),
                  ln1_b.reshape(1, hh), W2)
    acc2 = scat_kernel(m2s, src2d, dst2d)
    logits = _tc_final(acc2, m2s, dinv, b2.reshape(1, hh),
                       ln2_g.reshape(1, hh), ln2_b.reshape(1, hh),
                       batch.reshape(n, 1), Wc, bc.reshape(1, -1), n_seg)
    return logits


# speedup vs baseline: 1.6278x; 1.1037x over previous
import functools

import jax
import jax.numpy as jnp
from jax import lax
from jax.experimental import pallas as pl
from jax.experimental.pallas import tpu as pltpu
from jax.experimental.pallas import tpu_sc as plsc

NC = 2
NS = 16
LANES = 16
CHUNK = 128
PHASES = 4
GRAN = 8 * PHASES
SC0_FRAC = 0.8


def _pad_up(v, m):
    return -(-v // m) * m


def _split_rows(e):
    total_pw = _pad_up(_pad_up(e, NS * CHUNK) // (NS * CHUNK), GRAN)
    pw0 = int(round(total_pw * SC0_FRAC / GRAN)) * GRAN
    pw0 = min(max(pw0, GRAN), total_pw - GRAN)
    pw1 = total_pw - pw0
    return pw0, pw1


def _make_deg_kernel(n_pad, pw0, pw1):
    mesh = plsc.VectorSubcoreMesh(core_axis_name="c", subcore_axis_name="s")
    nw = NC * NS

    @functools.partial(
        pl.kernel,
        out_type=jax.ShapeDtypeStruct((nw, n_pad), jnp.float32),
        mesh=mesh,
        scratch_types=[
            pltpu.VMEM((n_pad,), jnp.float32),
            pltpu.VMEM((pw0, CHUNK), jnp.int32),
        ],
        compiler_params=pltpu.CompilerParams(needs_layout_passes=False),
    )
    def deg_kernel(dst_hbm, out_hbm, hist, dstv):
        cid = lax.axis_index("c")
        sid = lax.axis_index("s")
        wid = sid * NC + cid
        my_pw = jnp.where(cid == 0, pw0, pw1)
        row_base = pl.multiple_of(
            jnp.where(cid == 0, sid * pw0, NS * pw0 + sid * pw1), 8
        )
        z16 = jnp.zeros((LANES,), jnp.float32)

        def zb(i, c):
            hist[pl.ds(i * LANES, LANES)] = z16
            return c

        lax.fori_loop(0, n_pad // LANES, zb, 0)
        pltpu.sync_copy(dst_hbm.at[pl.ds(row_base, pw0)], dstv)
        ones16 = jnp.ones((LANES,), jnp.float32)
        groups = CHUNK // LANES

        def eb(g, c):
            j = g // groups
            l = g - j * groups
            idx = dstv[j, pl.ds(l * LANES, LANES)]
            plsc.addupdate_scatter(hist, [idx], ones16)
            return c

        lax.fori_loop(0, my_pw * groups, eb, 0)
        pltpu.sync_copy(hist, out_hbm.at[wid])

    return deg_kernel


def _make_scatter_kernel(n_nodes, n_pad, pw0, pw1, nph0, nph1, h):
    mesh = plsc.VectorSubcoreMesh(core_axis_name="c", subcore_axis_name="s")
    rows_pt = n_pad // NS
    depth = 2
    ppw = pw0 // nph0
    ppw1 = pw1 // nph1
    assert pw0 % nph0 == 0 and pw1 % nph1 == 0 and ppw1 <= ppw
    assert ppw1 >= 2 * depth and ppw % depth == 0 and ppw1 % depth == 0
    assert ppw % 8 == 0 and ppw1 % 8 == 0

    @functools.partial(
        pl.kernel,
        out_type=jax.ShapeDtypeStruct((NC, n_pad, h), jnp.float32),
        mesh=mesh,
        scratch_types=[
            pltpu.VMEM((ppw, CHUNK), jnp.int32),
            pltpu.VMEM((ppw, CHUNK), jnp.int32),
            [pltpu.VMEM((CHUNK, h), jnp.float32)] * depth,
            pltpu.VMEM_SHARED((n_pad, h), jnp.float32),
            [pltpu.SemaphoreType.DMA] * depth,
            [pltpu.SemaphoreType.DMA] * depth,
        ],
        compiler_params=pltpu.CompilerParams(needs_layout_passes=False),
    )
    def scat_kernel(m_hbm, src_hbm, dst_hbm, out_hbm, srcv, dstv, bufs, acc,
                    gsem, ssem):
        cid = lax.axis_index("c")
        sid = lax.axis_index("s")
        my_ppw = jnp.where(cid == 0, ppw, ppw1)
        my_nph = jnp.where(cid == 0, nph0, nph1)
        row_base = jnp.where(cid == 0, sid * pw0, NS * pw0 + sid * pw1)
        z16 = jnp.zeros((LANES,), jnp.float32)
        per_row = h // LANES

        def zb(i, c):
            r = i // per_row
            q = i - r * per_row
            bufs[0][r, pl.ds(q * LANES, LANES)] = z16
            return c

        lax.fori_loop(0, CHUNK * per_row, zb, 0)
        for k in range(rows_pt // CHUNK):
            pltpu.sync_copy(
                bufs[0], acc.at[pl.ds(sid * rows_pt + k * CHUNK, CHUNK)]
            )
        plsc.subcore_barrier()

        def fire_gather(j, b):
            pltpu.async_copy(m_hbm.at[srcv.at[j]], bufs[b], gsem[b])

        def drain_gather(b):
            pltpu.make_async_copy(m_hbm.at[srcv.at[0]], bufs[b], gsem[b]).wait()

        def fire_scatter(j, b):
            pltpu.async_copy(bufs[b], acc.at[dstv.at[j]], ssem[b], add=True)

        def drain_scatter(b):
            pltpu.make_async_copy(bufs[b], acc.at[dstv.at[0]], ssem[b]).wait()

        for p in range(nph0):
            @pl.when(p < my_nph)
            def _phase():
                base = pl.multiple_of(row_base + p * my_ppw, 8)
                pltpu.sync_copy(src_hbm.at[pl.ds(base, ppw)], srcv)
                pltpu.sync_copy(dst_hbm.at[pl.ds(base, ppw)], dstv)

                for b in range(depth):
                    fire_gather(b, b)

                def body(i, c):
                    j0 = i * depth
                    for b in range(depth):
                        drain_gather(b)
                        fire_scatter(j0 + b, b)
                    for b in range(depth):
                        drain_scatter(b)
                        fire_gather(j0 + depth + b, b)
                    return c

                lax.fori_loop(0, my_ppw // depth - 1, body, 0)
                j0 = my_ppw - depth
                for b in range(depth):
                    drain_gather(b)
                    fire_scatter(j0 + b, b)
                for b in range(depth):
                    drain_scatter(b)

        plsc.subcore_barrier()
        pltpu.sync_copy(
            acc.at[pl.ds(sid * rows_pt, rows_pt)],
            out_hbm.at[cid, pl.ds(sid * rows_pt, rows_pt)],
        )

    return scat_kernel


def _tc_prep(dp, x, w1):
    n, _ = x.shape
    hh = w1.shape[0]
    nw = dp.shape[0]

    def body(dp_ref, x_ref, w_ref, m_ref, dinv_ref):
        ones = jnp.ones((nw, 1), jnp.float32)
        deg = lax.dot_general(
            dp_ref[:, :n], ones, (((0,), (0,)), ((), ())),
            preferred_element_type=jnp.float32,
        )
        dinv = lax.rsqrt(deg + 1.0)
        m = lax.dot_general(
            x_ref[...], w_ref[...], (((1,), (1,)), ((), ())),
            preferred_element_type=jnp.float32,
        )
        m_ref[...] = m * dinv
        dinv_ref[...] = dinv

    return pl.pallas_call(
        body,
        out_shape=(
            jax.ShapeDtypeStruct((n, hh), jnp.float32),
            jax.ShapeDtypeStruct((n, 1), jnp.float32),
        ),
    )(dp, x, w1)


def _norm_act(acc_ref, m_ref, dinv_ref, b_ref, g_ref, beta_ref, n):
    agg = acc_ref[0, :n, :] + acc_ref[1, :n, :] + m_ref[...]
    pre = agg * dinv_ref[...] + b_ref[...]
    mu = jnp.mean(pre, axis=1, keepdims=True)
    var = jnp.mean((pre - mu) ** 2, axis=1, keepdims=True)
    y = (pre - mu) * lax.rsqrt(var + 1e-5) * g_ref[...] + beta_ref[...]
    return jnp.where(y > 0, y, jnp.exp(jnp.minimum(y, 0.0)) - 1.0)


def _tc_mid(acc, m, dinv, b, g, beta, w2):
    n, hh = m.shape

    def body(acc_ref, m_ref, dinv_ref, b_ref, g_ref, beta_ref, w_ref, out_ref):
        hcur = _norm_act(acc_ref, m_ref, dinv_ref, b_ref, g_ref, beta_ref, n)
        m2 = lax.dot_general(
            hcur, w_ref[...], (((1,), (1,)), ((), ())),
            preferred_element_type=jnp.float32,
        )
        out_ref[...] = m2 * dinv_ref[...]

    return pl.pallas_call(
        body,
        out_shape=jax.ShapeDtypeStruct((n, hh), jnp.float32),
    )(acc, m, dinv, b, g, beta, w2)


def _tc_final(acc, m, dinv, b, g, beta, batch2d, wc, bc, n_seg):
    n, hh = m.shape
    c = wc.shape[0]

    def body(acc_ref, m_ref, dinv_ref, b_ref, g_ref, beta_ref, batch_ref,
             wc_ref, bc_ref, out_ref):
        hcur = _norm_act(acc_ref, m_ref, dinv_ref, b_ref, g_ref, beta_ref, n)
        seg_iota = lax.broadcasted_iota(jnp.int32, (n, n_seg), 1)
        onehot = (batch_ref[...] == seg_iota).astype(jnp.float32)
        sums = lax.dot_general(
            onehot, hcur, (((0,), (0,)), ((), ())),
            preferred_element_type=jnp.float32,
        )
        cnt = lax.dot_general(
            onehot, jnp.ones((n, 1), jnp.float32), (((0,), (0,)), ((), ())),
            preferred_element_type=jnp.float32,
        )
        gpool = sums / jnp.maximum(cnt, 1.0)
        out_ref[...] = lax.dot_general(
            gpool, wc_ref[...], (((1,), (1,)), ((), ())),
            preferred_element_type=jnp.float32,
        ) + bc_ref[...]

    return pl.pallas_call(
        body,
        out_shape=jax.ShapeDtypeStruct((n_seg, c), jnp.float32),
    )(acc, m, dinv, b, g, beta, batch2d, wc, bc)


def kernel(x, edge_index, batch, W1, b1, W2, b2, ln1_g, ln1_b, ln2_g, ln2_b,
           Wc, bc):
    n, d = x.shape
    hh = W1.shape[0]
    e = edge_index.shape[1]
    n_seg = 64

    pw0, pw1 = _split_rows(e)
    tot_rows = NS * (pw0 + pw1)
    pad_rows = tot_rows + pw0
    ep = pad_rows * CHUNK
    n_pad = _pad_up(n + 1, NS * CHUNK)

    src = edge_index[0]
    dst = edge_index[1]
    src_p = jnp.concatenate([src, jnp.zeros((ep - e,), jnp.int32)])
    dst_p = jnp.concatenate([dst, jnp.full((ep - e,), n, jnp.int32)])
    src2d = src_p.reshape(pad_rows, CHUNK)
    dst2d = dst_p.reshape(pad_rows, CHUNK)

    total_pw = pw0 + pw1
    pw1s = max(8, _pad_up(total_pw // 10, 8))
    pw0s = total_pw - pw1s
    ppw0 = max(q for q in range(8, 57, 8) if pw0s % q == 0)
    nph0 = pw0s // ppw0
    nph1 = 1 if pw1s <= ppw0 else pw1s // ppw0

    deg_kernel = _make_deg_kernel(n_pad, pw0, pw1)
    scat_kernel = _make_scatter_kernel(n, n_pad, pw0s, pw1s, nph0, nph1, hh)

    dp = deg_kernel(dst2d)
    m1s, dinv = _tc_prep(dp, x, W1)
    acc1 = scat_kernel(m1s, src2d, dst2d)
    m2s = _tc_mid(acc1, m1s, dinv, b1.reshape(1, hh), ln1_g.reshape(1, hh),
                  ln1_b.reshape(1, hh), W2)
    acc2 = scat_kernel(m2s, src2d, dst2d)
    logits = _tc_final(acc2, m2s, dinv, b2.reshape(1, hh),
                       ln2_g.reshape(1, hh), ln2_b.reshape(1, hh),
                       batch.reshape(n, 1), Wc, bc.reshape(1, -1), n_seg)
    return logits
